# baseline (speedup 1.0000x reference)
import functools

import jax
import jax.numpy as jnp
from jax import lax
from jax.experimental import pallas as pl
from jax.experimental.pallas import tpu as pltpu
from jax.experimental.pallas import tpu_sc as plsc

N = 10000
D = 128
E = 320000
G = 16

NP = N + 8
WD = 144
BN = 1112
GRID = NP // BN

NTILES = 32
CHUNK = 128
CPT0 = 72
CPT1 = 86
EPT0 = CPT0 * CHUNK
EPT1 = CPT1 * CHUNK
EPAD = 16 * (EPT0 + EPT1)
NEG = -1e30


def _pre_body(x_ref, w_ref, asr, adr, he_ref, asc_ref, adc_ref, m_ref, sm):
    g = pl.program_id(0)
    h = jnp.dot(x_ref[:], w_ref[:], preferred_element_type=jnp.float32)
    he_ref[:, pl.ds(0, 128)] = h
    lane = lax.broadcasted_iota(jnp.int32, (BN, 16), 1)
    he_ref[:, pl.ds(128, 16)] = jnp.where(lane == 0, 1.0, 0.0).astype(jnp.float32)
    a_s = jnp.sum(h * asr[:], axis=1, keepdims=True)
    a_d = jnp.sum(h * adr[:], axis=1, keepdims=True)
    rid = g * BN + lax.broadcasted_iota(jnp.int32, (BN, 1), 0)
    a_s = jnp.where(rid < N, a_s, NEG)
    a_d = jnp.where(rid < N, a_d, NEG)
    asc_ref[:] = a_s
    adc_ref[:] = a_d
    c0 = jnp.where(g == 0, NEG, sm[0])
    c1 = jnp.where(g == 0, NEG, sm[1])
    sm[0] = jnp.maximum(c0, jnp.max(a_s))
    sm[1] = jnp.maximum(c1, jnp.max(a_d))

    @pl.when(g == GRID - 1)
    def _():
        mm = sm[0] + sm[1]
        mv = jnp.where(mm > 0, mm, 0.2 * mm)
        m_ref[:] = jnp.full((1, 128), mv, dtype=jnp.float32)


def _pre(x_pad, W, att_src, att_dst):
    return pl.pallas_call(
        _pre_body,
        grid=(GRID,),
        in_specs=[
            pl.BlockSpec((BN, 128), lambda g: (g, 0)),
            pl.BlockSpec((128, 128), lambda g: (0, 0)),
            pl.BlockSpec((1, 128), lambda g: (0, 0)),
            pl.BlockSpec((1, 128), lambda g: (0, 0)),
        ],
        out_specs=[
            pl.BlockSpec((BN, WD), lambda g: (g, 0)),
            pl.BlockSpec((BN, 1), lambda g: (g, 0)),
            pl.BlockSpec((BN, 1), lambda g: (g, 0)),
            pl.BlockSpec((1, 128), lambda g: (0, 0)),
        ],
        out_shape=[
            jax.ShapeDtypeStruct((NP, WD), jnp.float32),
            jax.ShapeDtypeStruct((NP, 1), jnp.float32),
            jax.ShapeDtypeStruct((NP, 1), jnp.float32),
            jax.ShapeDtypeStruct((1, 128), jnp.float32),
        ],
        scratch_shapes=[pltpu.SMEM((2,), jnp.float32)],
    )(x_pad, W, att_src, att_dst)


def _sc_body(he_hbm, as_hbm, ad_hbm, m_hbm, src_hbm, dst_hbm, part_hbm,
             acc, as_v, ad_v, m_v, src_v, dst_v, rows_v, ex_v, sem):
    cid = lax.axis_index("c")
    sid = lax.axis_index("s")
    t = cid * 16 + sid

    pltpu.sync_copy(as_hbm, as_v)
    pltpu.sync_copy(ad_hbm, ad_v)
    pltpu.sync_copy(m_hbm, m_v)

    def zbody(i, carry):
        for j in range(WD // 16):
            rows_v[i, pl.ds(j * 16, 16)] = jnp.zeros((16,), jnp.float32)
        return carry

    lax.fori_loop(0, CHUNK, zbody, 0)

    row0 = sid * 624

    @pl.when(sid < 15)
    def _():
        for k in range(4):
            pltpu.sync_copy(rows_v.at[pl.ds(0, 128)],
                            acc.at[pl.ds(row0 + k * 128, 128)])
        pltpu.sync_copy(rows_v.at[pl.ds(0, 112)],
                        acc.at[pl.ds(row0 + 512, 112)])

    @pl.when(sid == 15)
    def _():
        for k in range(5):
            pltpu.sync_copy(rows_v.at[pl.ds(0, 128)],
                            acc.at[pl.ds(9360 + k * 128, 128)])
        pltpu.sync_copy(rows_v.at[pl.ds(0, 8)],
                        acc.at[pl.ds(10000, 8)])

    plsc.subcore_barrier()

    base = jnp.where(cid == 0, sid * EPT0, 16 * EPT0 + sid * EPT1)
    nchunks = jnp.where(cid == 0, CPT0, CPT1)
    m0 = m_v[pl.ds(0, 16)][0]

    def chunk(c, carry):
        off = base + c * CHUNK
        pltpu.sync_copy(src_hbm.at[pl.ds(off, CHUNK)], src_v)
        pltpu.sync_copy(dst_hbm.at[pl.ds(off, CHUNK)], dst_v)
        cp = pltpu.async_copy(he_hbm.at[src_v], rows_v, sem)
        for k in range(CHUNK // 16):
            sv = src_v[pl.ds(k * 16, 16)]
            dv = dst_v[pl.ds(k * 16, 16)]
            e = plsc.load_gather(as_v, [sv]) + plsc.load_gather(ad_v, [dv])
            e = jnp.where(e > 0, e, e * 0.2)
            ex_v[pl.ds(k * 16, 16)] = jnp.exp(e - m0)
        cp.wait()

        def scale(k, carry2):
            exv = ex_v[pl.ds(k * 16, 16)]
            for l in range(16):
                w = exv[l]
                i = k * 16 + l
                for j in range(WD // 16):
                    rows_v[i, pl.ds(j * 16, 16)] = (
                        rows_v[i, pl.ds(j * 16, 16)] * w)
            return carry2

        lax.fori_loop(0, CHUNK // 16, scale, 0)
        pltpu.sync_copy(rows_v, acc.at[dst_v], add=True)
        return carry

    lax.fori_loop(0, nchunks, chunk, 0)
    plsc.subcore_barrier()

    @pl.when(sid < 15)
    def _():
        pltpu.sync_copy(acc.at[pl.ds(row0, 624)],
                        part_hbm.at[cid, pl.ds(row0, 624)])

    @pl.when(sid == 15)
    def _():
        pltpu.sync_copy(acc.at[pl.ds(9360, 648)],
                        part_hbm.at[cid, pl.ds(9360, 648)])


def _sc_edge_pass(he, a_src, a_dst, m16, src_p, dst_p):
    mesh = plsc.VectorSubcoreMesh(core_axis_name="c", subcore_axis_name="s")
    k = functools.partial(
        pl.kernel,
        out_type=jax.ShapeDtypeStruct((2, NP, WD), jnp.float32),
        mesh=mesh,
        compiler_params=pltpu.CompilerParams(
            needs_layout_passes=False, use_tc_tiling_on_sc=False),
        scratch_types=[
            pltpu.VMEM_SHARED((NP, WD), jnp.float32),
            pltpu.VMEM((NP,), jnp.float32),
            pltpu.VMEM((NP,), jnp.float32),
            pltpu.VMEM((16,), jnp.float32),
            pltpu.VMEM((CHUNK,), jnp.int32),
            pltpu.VMEM((CHUNK,), jnp.int32),
            pltpu.VMEM((CHUNK, WD), jnp.float32),
            pltpu.VMEM((CHUNK,), jnp.float32),
            pltpu.SemaphoreType.DMA,
        ],
    )(_sc_body)
    return k(he, a_src, a_dst, m16, src_p, dst_p)


def _epi_body(part_ref, he_ref, asr, adr, m_ref, b_ref, bat_ref, out_ref):
    g = pl.program_id(0)
    acc = part_ref[0, :, pl.ds(0, 128)] + part_ref[1, :, pl.ds(0, 128)]
    den = part_ref[0, :, pl.ds(128, 1)] + part_ref[1, :, pl.ds(128, 1)]
    es = asr[:] + adr[:]
    es = jnp.where(es > 0, es, es * 0.2)
    exs = jnp.exp(es - m_ref[0, 0])
    tot = acc + exs * he_ref[:, pl.ds(0, 128)]
    dent = den + exs + 1e-16
    r = jnp.maximum(tot / dent + b_ref[:], 0.0)
    cur = jnp.where(g == 0, jnp.full((G, 128), -jnp.inf, jnp.float32), out_ref[:])
    rows = []
    for gg in range(G):
        rm = jnp.where(bat_ref[:] == gg, r, -jnp.inf)
        rows.append(jnp.max(rm, axis=0, keepdims=True))
    cur = jnp.maximum(cur, jnp.concatenate(rows, axis=0))

    @pl.when(g < GRID - 1)
    def _():
        out_ref[:] = cur

    @pl.when(g == GRID - 1)
    def _():
        out_ref[:] = jnp.where(jnp.isfinite(cur), cur, 0.0)


def _epilogue(part, he, a_src, a_dst, m128, bias, batch2d):
    return pl.pallas_call(
        _epi_body,
        grid=(GRID,),
        in_specs=[
            pl.BlockSpec((2, BN, WD), lambda g: (0, g, 0)),
            pl.BlockSpec((BN, WD), lambda g: (g, 0)),
            pl.BlockSpec((BN, 1), lambda g: (g, 0)),
            pl.BlockSpec((BN, 1), lambda g: (g, 0)),
            pl.BlockSpec((1, 128), lambda g: (0, 0)),
            pl.BlockSpec((1, 128), lambda g: (0, 0)),
            pl.BlockSpec((BN, 1), lambda g: (g, 0)),
        ],
        out_specs=pl.BlockSpec((G, 128), lambda g: (0, 0)),
        out_shape=jax.ShapeDtypeStruct((G, 128), jnp.float32),
    )(part, he, a_src, a_dst, m128, bias, batch2d)


def kernel(x, edge_index, batch, W, att_src, att_dst, bias):
    x_pad = jnp.concatenate([x, jnp.zeros((NP - N, D), jnp.float32)], axis=0)
    pad = jnp.full((EPAD - E,), N, jnp.int32)
    src_p = jnp.concatenate([edge_index[0], pad])
    dst_p = jnp.concatenate([edge_index[1], pad])
    batch2d = jnp.concatenate([batch, jnp.full((NP - N,), G, jnp.int32)]).reshape(NP, 1)

    he, a_src2d, a_dst2d, m128 = _pre(
        x_pad, W, att_src.reshape(1, 128), att_dst.reshape(1, 128))
    part = _sc_edge_pass(he, a_src2d.reshape(NP), a_dst2d.reshape(NP),
                         m128[0, :16], src_p, dst_p)
    return _epilogue(part, he, a_src2d, a_dst2d, m128,
                     bias.reshape(1, 128), batch2d)


# speedup vs baseline: 1.0895x; 1.0895x over previous
import functools

import jax
import jax.numpy as jnp
from jax import lax
from jax.experimental import pallas as pl
from jax.experimental.pallas import tpu as pltpu
from jax.experimental.pallas import tpu_sc as plsc

N = 10000
D = 128
E = 320000
G = 16

NP = N + 8
WD = 144
BN = 1112
GRID = NP // BN

NTILES = 32
CHUNK = 128
CPT0 = 86
CPT1 = 72
EPT0 = CPT0 * CHUNK
EPT1 = CPT1 * CHUNK
EPAD = 16 * (EPT0 + EPT1)
NEG = -1e30


def _pre_body(x_ref, w_ref, asr, adr, he_ref, asc_ref, adc_ref, m_ref, sm):
    g = pl.program_id(0)
    h = jnp.dot(x_ref[:], w_ref[:], preferred_element_type=jnp.float32)
    he_ref[:, pl.ds(0, 128)] = h
    lane = lax.broadcasted_iota(jnp.int32, (BN, 16), 1)
    he_ref[:, pl.ds(128, 16)] = jnp.where(lane == 0, 1.0, 0.0).astype(jnp.float32)
    a_s = jnp.sum(h * asr[:], axis=1, keepdims=True)
    a_d = jnp.sum(h * adr[:], axis=1, keepdims=True)
    rid = g * BN + lax.broadcasted_iota(jnp.int32, (BN, 1), 0)
    a_s = jnp.where(rid < N, a_s, NEG)
    a_d = jnp.where(rid < N, a_d, NEG)
    asc_ref[:] = a_s
    adc_ref[:] = a_d
    c0 = jnp.where(g == 0, NEG, sm[0])
    c1 = jnp.where(g == 0, NEG, sm[1])
    sm[0] = jnp.maximum(c0, jnp.max(a_s))
    sm[1] = jnp.maximum(c1, jnp.max(a_d))

    @pl.when(g == GRID - 1)
    def _():
        mm = sm[0] + sm[1]
        mv = jnp.where(mm > 0, mm, 0.2 * mm)
        m_ref[:] = jnp.full((1, 128), mv, dtype=jnp.float32)


def _pre(x_pad, W, att_src, att_dst):
    return pl.pallas_call(
        _pre_body,
        grid=(GRID,),
        in_specs=[
            pl.BlockSpec((BN, 128), lambda g: (g, 0)),
            pl.BlockSpec((128, 128), lambda g: (0, 0)),
            pl.BlockSpec((1, 128), lambda g: (0, 0)),
            pl.BlockSpec((1, 128), lambda g: (0, 0)),
        ],
        out_specs=[
            pl.BlockSpec((BN, WD), lambda g: (g, 0)),
            pl.BlockSpec((BN, 1), lambda g: (g, 0)),
            pl.BlockSpec((BN, 1), lambda g: (g, 0)),
            pl.BlockSpec((1, 128), lambda g: (0, 0)),
        ],
        out_shape=[
            jax.ShapeDtypeStruct((NP, WD), jnp.float32),
            jax.ShapeDtypeStruct((NP, 1), jnp.float32),
            jax.ShapeDtypeStruct((NP, 1), jnp.float32),
            jax.ShapeDtypeStruct((1, 128), jnp.float32),
        ],
        scratch_shapes=[pltpu.SMEM((2,), jnp.float32)],
    )(x_pad, W, att_src, att_dst)


def _sc_body(he_hbm, as_hbm, ad_hbm, m_hbm, src_hbm, dst_hbm, part_hbm,
             acc, as_v, ad_v, m_v, src_v, dst_v, rows_v, ex_v, sem):
    cid = lax.axis_index("c")
    sid = lax.axis_index("s")
    t = cid * 16 + sid

    pltpu.sync_copy(as_hbm, as_v)
    pltpu.sync_copy(ad_hbm, ad_v)
    pltpu.sync_copy(m_hbm, m_v)

    def zbody(i, carry):
        for j in range(WD // 16):
            rows_v[i, pl.ds(j * 16, 16)] = jnp.zeros((16,), jnp.float32)
        return carry

    lax.fori_loop(0, CHUNK, zbody, 0)

    row0 = sid * 624

    @pl.when(sid < 15)
    def _():
        for k in range(4):
            pltpu.sync_copy(rows_v.at[pl.ds(0, 128)],
                            acc.at[pl.ds(row0 + k * 128, 128)])
        pltpu.sync_copy(rows_v.at[pl.ds(0, 112)],
                        acc.at[pl.ds(row0 + 512, 112)])

    @pl.when(sid == 15)
    def _():
        for k in range(5):
            pltpu.sync_copy(rows_v.at[pl.ds(0, 128)],
                            acc.at[pl.ds(9360 + k * 128, 128)])
        pltpu.sync_copy(rows_v.at[pl.ds(0, 8)],
                        acc.at[pl.ds(10000, 8)])

    plsc.subcore_barrier()

    base = jnp.where(cid == 0, sid * EPT0, 16 * EPT0 + sid * EPT1)
    nchunks = jnp.where(cid == 0, CPT0, CPT1)
    m0 = m_v[pl.ds(0, 16)][0]

    def chunk(c, carry):
        off = base + c * CHUNK
        pltpu.sync_copy(src_hbm.at[pl.ds(off, CHUNK)], src_v)
        pltpu.sync_copy(dst_hbm.at[pl.ds(off, CHUNK)], dst_v)
        cp = pltpu.async_copy(he_hbm.at[src_v], rows_v, sem)
        for k in range(CHUNK // 16):
            sv = src_v[pl.ds(k * 16, 16)]
            dv = dst_v[pl.ds(k * 16, 16)]
            e = plsc.load_gather(as_v, [sv]) + plsc.load_gather(ad_v, [dv])
            e = jnp.where(e > 0, e, e * 0.2)
            ex_v[pl.ds(k * 16, 16)] = jnp.exp(e - m0)
        cp.wait()

        def scale(k, carry2):
            exv = ex_v[pl.ds(k * 16, 16)]
            for l in range(16):
                w = exv[l]
                i = k * 16 + l
                for j in range(WD // 16):
                    rows_v[i, pl.ds(j * 16, 16)] = (
                        rows_v[i, pl.ds(j * 16, 16)] * w)
            return carry2

        lax.fori_loop(0, CHUNK // 16, scale, 0)
        pltpu.sync_copy(rows_v, acc.at[dst_v], add=True)
        return carry

    lax.fori_loop(0, nchunks, chunk, 0)
    plsc.subcore_barrier()

    @pl.when(sid < 15)
    def _():
        pltpu.sync_copy(acc.at[pl.ds(row0, 624)],
                        part_hbm.at[cid, pl.ds(row0, 624)])

    @pl.when(sid == 15)
    def _():
        pltpu.sync_copy(acc.at[pl.ds(9360, 648)],
                        part_hbm.at[cid, pl.ds(9360, 648)])


def _sc_edge_pass(he, a_src, a_dst, m16, src_p, dst_p):
    mesh = plsc.VectorSubcoreMesh(core_axis_name="c", subcore_axis_name="s")
    k = functools.partial(
        pl.kernel,
        out_type=jax.ShapeDtypeStruct((2, NP, WD), jnp.float32),
        mesh=mesh,
        compiler_params=pltpu.CompilerParams(
            needs_layout_passes=False, use_tc_tiling_on_sc=False),
        scratch_types=[
            pltpu.VMEM_SHARED((NP, WD), jnp.float32),
            pltpu.VMEM((NP,), jnp.float32),
            pltpu.VMEM((NP,), jnp.float32),
            pltpu.VMEM((16,), jnp.float32),
            pltpu.VMEM((CHUNK,), jnp.int32),
            pltpu.VMEM((CHUNK,), jnp.int32),
            pltpu.VMEM((CHUNK, WD), jnp.float32),
            pltpu.VMEM((CHUNK,), jnp.float32),
            pltpu.SemaphoreType.DMA,
        ],
    )(_sc_body)
    return k(he, a_src, a_dst, m16, src_p, dst_p)


def _epi_body(part_ref, he_ref, asr, adr, m_ref, b_ref, bat_ref, out_ref):
    g = pl.program_id(0)
    acc = part_ref[0, :, pl.ds(0, 128)] + part_ref[1, :, pl.ds(0, 128)]
    den = part_ref[0, :, pl.ds(128, 1)] + part_ref[1, :, pl.ds(128, 1)]
    es = asr[:] + adr[:]
    es = jnp.where(es > 0, es, es * 0.2)
    exs = jnp.exp(es - m_ref[0, 0])
    tot = acc + exs * he_ref[:, pl.ds(0, 128)]
    dent = den + exs + 1e-16
    r = jnp.maximum(tot / dent + b_ref[:], 0.0)
    cur = jnp.where(g == 0, jnp.full((G, 128), -jnp.inf, jnp.float32), out_ref[:])
    rows = []
    for gg in range(G):
        rm = jnp.where(bat_ref[:] == gg, r, -jnp.inf)
        rows.append(jnp.max(rm, axis=0, keepdims=True))
    cur = jnp.maximum(cur, jnp.concatenate(rows, axis=0))

    @pl.when(g < GRID - 1)
    def _():
        out_ref[:] = cur

    @pl.when(g == GRID - 1)
    def _():
        out_ref[:] = jnp.where(jnp.isfinite(cur), cur, 0.0)


def _epilogue(part, he, a_src, a_dst, m128, bias, batch2d):
    return pl.pallas_call(
        _epi_body,
        grid=(GRID,),
        in_specs=[
            pl.BlockSpec((2, BN, WD), lambda g: (0, g, 0)),
            pl.BlockSpec((BN, WD), lambda g: (g, 0)),
            pl.BlockSpec((BN, 1), lambda g: (g, 0)),
            pl.BlockSpec((BN, 1), lambda g: (g, 0)),
            pl.BlockSpec((1, 128), lambda g: (0, 0)),
            pl.BlockSpec((1, 128), lambda g: (0, 0)),
            pl.BlockSpec((BN, 1), lambda g: (g, 0)),
        ],
        out_specs=pl.BlockSpec((G, 128), lambda g: (0, 0)),
        out_shape=jax.ShapeDtypeStruct((G, 128), jnp.float32),
    )(part, he, a_src, a_dst, m128, bias, batch2d)


def kernel(x, edge_index, batch, W, att_src, att_dst, bias):
    x_pad = jnp.concatenate([x, jnp.zeros((NP - N, D), jnp.float32)], axis=0)
    pad = jnp.full((EPAD - E,), N, jnp.int32)
    src_p = jnp.concatenate([edge_index[0], pad])
    dst_p = jnp.concatenate([edge_index[1], pad])
    batch2d = jnp.concatenate([batch, jnp.full((NP - N,), G, jnp.int32)]).reshape(NP, 1)

    he, a_src2d, a_dst2d, m128 = _pre(
        x_pad, W, att_src.reshape(1, 128), att_dst.reshape(1, 128))
    part = _sc_edge_pass(he, a_src2d.reshape(NP), a_dst2d.reshape(NP),
                         m128[0, :16], src_p, dst_p)
    return _epilogue(part, he, a_src2d, a_dst2d, m128,
                     bias.reshape(1, 128), batch2d)


# speedup vs baseline: 1.1285x; 1.0358x over previous
import functools

import jax
import jax.numpy as jnp
from jax import lax
from jax.experimental import pallas as pl
from jax.experimental.pallas import tpu as pltpu
from jax.experimental.pallas import tpu_sc as plsc

N = 10000
D = 128
E = 320000
G = 16

NP = N + 8
WD = 144
BN = 1112
GRID = NP // BN

NTILES = 32
CHUNK = 128
CPT0 = 91
CPT1 = 67
EPT0 = CPT0 * CHUNK
EPT1 = CPT1 * CHUNK
EPAD = 16 * (EPT0 + EPT1)
NEG = -1e30


def _pre_body(x_ref, w_ref, asr, adr, he_ref, asc_ref, adc_ref, m_ref, sm):
    g = pl.program_id(0)
    h = jnp.dot(x_ref[:], w_ref[:], preferred_element_type=jnp.float32)
    he_ref[:, pl.ds(0, 128)] = h
    lane = lax.broadcasted_iota(jnp.int32, (BN, 16), 1)
    he_ref[:, pl.ds(128, 16)] = jnp.where(lane == 0, 1.0, 0.0).astype(jnp.float32)
    a_s = jnp.sum(h * asr[:], axis=1, keepdims=True)
    a_d = jnp.sum(h * adr[:], axis=1, keepdims=True)
    rid = g * BN + lax.broadcasted_iota(jnp.int32, (BN, 1), 0)
    a_s = jnp.where(rid < N, a_s, NEG)
    a_d = jnp.where(rid < N, a_d, NEG)
    asc_ref[:] = a_s
    adc_ref[:] = a_d
    c0 = jnp.where(g == 0, NEG, sm[0])
    c1 = jnp.where(g == 0, NEG, sm[1])
    sm[0] = jnp.maximum(c0, jnp.max(a_s))
    sm[1] = jnp.maximum(c1, jnp.max(a_d))

    @pl.when(g == GRID - 1)
    def _():
        mm = sm[0] + sm[1]
        mv = jnp.where(mm > 0, mm, 0.2 * mm)
        m_ref[:] = jnp.full((1, 128), mv, dtype=jnp.float32)


def _pre(x_pad, W, att_src, att_dst):
    return pl.pallas_call(
        _pre_body,
        grid=(GRID,),
        in_specs=[
            pl.BlockSpec((BN, 128), lambda g: (g, 0)),
            pl.BlockSpec((128, 128), lambda g: (0, 0)),
            pl.BlockSpec((1, 128), lambda g: (0, 0)),
            pl.BlockSpec((1, 128), lambda g: (0, 0)),
        ],
        out_specs=[
            pl.BlockSpec((BN, WD), lambda g: (g, 0)),
            pl.BlockSpec((BN, 1), lambda g: (g, 0)),
            pl.BlockSpec((BN, 1), lambda g: (g, 0)),
            pl.BlockSpec((1, 128), lambda g: (0, 0)),
        ],
        out_shape=[
            jax.ShapeDtypeStruct((NP, WD), jnp.float32),
            jax.ShapeDtypeStruct((NP, 1), jnp.float32),
            jax.ShapeDtypeStruct((NP, 1), jnp.float32),
            jax.ShapeDtypeStruct((1, 128), jnp.float32),
        ],
        scratch_shapes=[pltpu.SMEM((2,), jnp.float32)],
    )(x_pad, W, att_src, att_dst)


def _sc_body(he_hbm, as_hbm, ad_hbm, m_hbm, src_hbm, dst_hbm, part_hbm,
             acc, as_v, ad_v, m_v, src_v, dst_v, rows_v, ex_v, sem):
    cid = lax.axis_index("c")
    sid = lax.axis_index("s")
    t = cid * 16 + sid

    pltpu.sync_copy(as_hbm, as_v)
    pltpu.sync_copy(ad_hbm, ad_v)
    pltpu.sync_copy(m_hbm, m_v)

    def zbody(i, carry):
        for j in range(WD // 16):
            rows_v[i, pl.ds(j * 16, 16)] = jnp.zeros((16,), jnp.float32)
        return carry

    lax.fori_loop(0, CHUNK, zbody, 0)

    row0 = sid * 624

    @pl.when(sid < 15)
    def _():
        for k in range(4):
            pltpu.sync_copy(rows_v.at[pl.ds(0, 128)],
                            acc.at[pl.ds(row0 + k * 128, 128)])
        pltpu.sync_copy(rows_v.at[pl.ds(0, 112)],
                        acc.at[pl.ds(row0 + 512, 112)])

    @pl.when(sid == 15)
    def _():
        for k in range(5):
            pltpu.sync_copy(rows_v.at[pl.ds(0, 128)],
                            acc.at[pl.ds(9360 + k * 128, 128)])
        pltpu.sync_copy(rows_v.at[pl.ds(0, 8)],
                        acc.at[pl.ds(10000, 8)])

    plsc.subcore_barrier()

    base = jnp.where(cid == 0, sid * EPT0, 16 * EPT0 + sid * EPT1)
    nchunks = jnp.where(cid == 0, CPT0, CPT1)
    m0 = m_v[pl.ds(0, 16)][0]

    def chunk(c, carry):
        off = base + c * CHUNK
        pltpu.sync_copy(src_hbm.at[pl.ds(off, CHUNK)], src_v)
        pltpu.sync_copy(dst_hbm.at[pl.ds(off, CHUNK)], dst_v)
        cp = pltpu.async_copy(he_hbm.at[src_v], rows_v, sem)
        for k in range(CHUNK // 16):
            sv = src_v[pl.ds(k * 16, 16)]
            dv = dst_v[pl.ds(k * 16, 16)]
            e = plsc.load_gather(as_v, [sv]) + plsc.load_gather(ad_v, [dv])
            e = jnp.where(e > 0, e, e * 0.2)
            ex_v[pl.ds(k * 16, 16)] = jnp.exp(e - m0)
        cp.wait()

        def scale(k, carry2):
            exv = ex_v[pl.ds(k * 16, 16)]
            for l in range(16):
                w = exv[l]
                i = k * 16 + l
                for j in range(WD // 16):
                    rows_v[i, pl.ds(j * 16, 16)] = (
                        rows_v[i, pl.ds(j * 16, 16)] * w)
            return carry2

        lax.fori_loop(0, CHUNK // 16, scale, 0)
        pltpu.sync_copy(rows_v, acc.at[dst_v], add=True)
        return carry

    lax.fori_loop(0, nchunks, chunk, 0)
    plsc.subcore_barrier()

    @pl.when(sid < 15)
    def _():
        pltpu.sync_copy(acc.at[pl.ds(row0, 624)],
                        part_hbm.at[cid, pl.ds(row0, 624)])

    @pl.when(sid == 15)
    def _():
        pltpu.sync_copy(acc.at[pl.ds(9360, 648)],
                        part_hbm.at[cid, pl.ds(9360, 648)])


def _sc_edge_pass(he, a_src, a_dst, m16, src_p, dst_p):
    mesh = plsc.VectorSubcoreMesh(core_axis_name="c", subcore_axis_name="s")
    k = functools.partial(
        pl.kernel,
        out_type=jax.ShapeDtypeStruct((2, NP, WD), jnp.float32),
        mesh=mesh,
        compiler_params=pltpu.CompilerParams(
            needs_layout_passes=False, use_tc_tiling_on_sc=False),
        scratch_types=[
            pltpu.VMEM_SHARED((NP, WD), jnp.float32),
            pltpu.VMEM((NP,), jnp.float32),
            pltpu.VMEM((NP,), jnp.float32),
            pltpu.VMEM((16,), jnp.float32),
            pltpu.VMEM((CHUNK,), jnp.int32),
            pltpu.VMEM((CHUNK,), jnp.int32),
            pltpu.VMEM((CHUNK, WD), jnp.float32),
            pltpu.VMEM((CHUNK,), jnp.float32),
            pltpu.SemaphoreType.DMA,
        ],
    )(_sc_body)
    return k(he, a_src, a_dst, m16, src_p, dst_p)


def _epi_body(part_ref, he_ref, asr, adr, m_ref, b_ref, bat_ref, out_ref):
    g = pl.program_id(0)
    acc = part_ref[0, :, pl.ds(0, 128)] + part_ref[1, :, pl.ds(0, 128)]
    den = part_ref[0, :, pl.ds(128, 1)] + part_ref[1, :, pl.ds(128, 1)]
    es = asr[:] + adr[:]
    es = jnp.where(es > 0, es, es * 0.2)
    exs = jnp.exp(es - m_ref[0, 0])
    tot = acc + exs * he_ref[:, pl.ds(0, 128)]
    dent = den + exs + 1e-16
    r = jnp.maximum(tot / dent + b_ref[:], 0.0)
    cur = jnp.where(g == 0, jnp.full((G, 128), -jnp.inf, jnp.float32), out_ref[:])
    rows = []
    for gg in range(G):
        rm = jnp.where(bat_ref[:] == gg, r, -jnp.inf)
        rows.append(jnp.max(rm, axis=0, keepdims=True))
    cur = jnp.maximum(cur, jnp.concatenate(rows, axis=0))

    @pl.when(g < GRID - 1)
    def _():
        out_ref[:] = cur

    @pl.when(g == GRID - 1)
    def _():
        out_ref[:] = jnp.where(jnp.isfinite(cur), cur, 0.0)


def _epilogue(part, he, a_src, a_dst, m128, bias, batch2d):
    return pl.pallas_call(
        _epi_body,
        grid=(GRID,),
        in_specs=[
            pl.BlockSpec((2, BN, WD), lambda g: (0, g, 0)),
            pl.BlockSpec((BN, WD), lambda g: (g, 0)),
            pl.BlockSpec((BN, 1), lambda g: (g, 0)),
            pl.BlockSpec((BN, 1), lambda g: (g, 0)),
            pl.BlockSpec((1, 128), lambda g: (0, 0)),
            pl.BlockSpec((1, 128), lambda g: (0, 0)),
            pl.BlockSpec((BN, 1), lambda g: (g, 0)),
        ],
        out_specs=pl.BlockSpec((G, 128), lambda g: (0, 0)),
        out_shape=jax.ShapeDtypeStruct((G, 128), jnp.float32),
    )(part, he, a_src, a_dst, m128, bias, batch2d)


def kernel(x, edge_index, batch, W, att_src, att_dst, bias):
    x_pad = jnp.concatenate([x, jnp.zeros((NP - N, D), jnp.float32)], axis=0)
    pad = jnp.full((EPAD - E,), N, jnp.int32)
    src_p = jnp.concatenate([edge_index[0], pad])
    dst_p = jnp.concatenate([edge_index[1], pad])
    batch2d = jnp.concatenate([batch, jnp.full((NP - N,), G, jnp.int32)]).reshape(NP, 1)

    he, a_src2d, a_dst2d, m128 = _pre(
        x_pad, W, att_src.reshape(1, 128), att_dst.reshape(1, 128))
    part = _sc_edge_pass(he, a_src2d.reshape(NP), a_dst2d.reshape(NP),
                         m128[0, :16], src_p, dst_p)
    return _epilogue(part, he, a_src2d, a_dst2d, m128,
                     bias.reshape(1, 128), batch2d)


# speedup vs baseline: 1.1540x; 1.0226x over previous
import functools

import jax
import jax.numpy as jnp
from jax import lax
from jax.experimental import pallas as pl
from jax.experimental.pallas import tpu as pltpu
from jax.experimental.pallas import tpu_sc as plsc

N = 10000
D = 128
E = 320000
G = 16

NP = N + 8
WD = 144
BN = 1112
GRID = NP // BN

NTILES = 32
CHUNK = 128
CPT0 = 94
CPT1 = 64
EPT0 = CPT0 * CHUNK
EPT1 = CPT1 * CHUNK
EPAD = 16 * (EPT0 + EPT1)
NEG = -1e30


def _pre_body(x_ref, w_ref, asr, adr, he_ref, asc_ref, adc_ref, m_ref, sm):
    g = pl.program_id(0)
    h = jnp.dot(x_ref[:], w_ref[:], preferred_element_type=jnp.float32)
    he_ref[:, pl.ds(0, 128)] = h
    lane = lax.broadcasted_iota(jnp.int32, (BN, 16), 1)
    he_ref[:, pl.ds(128, 16)] = jnp.where(lane == 0, 1.0, 0.0).astype(jnp.float32)
    a_s = jnp.sum(h * asr[:], axis=1, keepdims=True)
    a_d = jnp.sum(h * adr[:], axis=1, keepdims=True)
    rid = g * BN + lax.broadcasted_iota(jnp.int32, (BN, 1), 0)
    a_s = jnp.where(rid < N, a_s, NEG)
    a_d = jnp.where(rid < N, a_d, NEG)
    asc_ref[:] = a_s
    adc_ref[:] = a_d
    c0 = jnp.where(g == 0, NEG, sm[0])
    c1 = jnp.where(g == 0, NEG, sm[1])
    sm[0] = jnp.maximum(c0, jnp.max(a_s))
    sm[1] = jnp.maximum(c1, jnp.max(a_d))

    @pl.when(g == GRID - 1)
    def _():
        mm = sm[0] + sm[1]
        mv = jnp.where(mm > 0, mm, 0.2 * mm)
        m_ref[:] = jnp.full((1, 128), mv, dtype=jnp.float32)


def _pre(x_pad, W, att_src, att_dst):
    return pl.pallas_call(
        _pre_body,
        grid=(GRID,),
        in_specs=[
            pl.BlockSpec((BN, 128), lambda g: (g, 0)),
            pl.BlockSpec((128, 128), lambda g: (0, 0)),
            pl.BlockSpec((1, 128), lambda g: (0, 0)),
            pl.BlockSpec((1, 128), lambda g: (0, 0)),
        ],
        out_specs=[
            pl.BlockSpec((BN, WD), lambda g: (g, 0)),
            pl.BlockSpec((BN, 1), lambda g: (g, 0)),
            pl.BlockSpec((BN, 1), lambda g: (g, 0)),
            pl.BlockSpec((1, 128), lambda g: (0, 0)),
        ],
        out_shape=[
            jax.ShapeDtypeStruct((NP, WD), jnp.float32),
            jax.ShapeDtypeStruct((NP, 1), jnp.float32),
            jax.ShapeDtypeStruct((NP, 1), jnp.float32),
            jax.ShapeDtypeStruct((1, 128), jnp.float32),
        ],
        scratch_shapes=[pltpu.SMEM((2,), jnp.float32)],
    )(x_pad, W, att_src, att_dst)


def _sc_body(he_hbm, as_hbm, ad_hbm, m_hbm, src_hbm, dst_hbm, part_hbm,
             acc, as_v, ad_v, m_v, src_v, dst_v, rows_v, ex_v, sem):
    cid = lax.axis_index("c")
    sid = lax.axis_index("s")
    t = cid * 16 + sid

    pltpu.sync_copy(as_hbm, as_v)
    pltpu.sync_copy(ad_hbm, ad_v)
    pltpu.sync_copy(m_hbm, m_v)

    def zbody(i, carry):
        for j in range(WD // 16):
            rows_v[i, pl.ds(j * 16, 16)] = jnp.zeros((16,), jnp.float32)
        return carry

    lax.fori_loop(0, CHUNK, zbody, 0)

    row0 = sid * 624

    @pl.when(sid < 15)
    def _():
        for k in range(4):
            pltpu.sync_copy(rows_v.at[pl.ds(0, 128)],
                            acc.at[pl.ds(row0 + k * 128, 128)])
        pltpu.sync_copy(rows_v.at[pl.ds(0, 112)],
                        acc.at[pl.ds(row0 + 512, 112)])

    @pl.when(sid == 15)
    def _():
        for k in range(5):
            pltpu.sync_copy(rows_v.at[pl.ds(0, 128)],
                            acc.at[pl.ds(9360 + k * 128, 128)])
        pltpu.sync_copy(rows_v.at[pl.ds(0, 8)],
                        acc.at[pl.ds(10000, 8)])

    plsc.subcore_barrier()

    base = jnp.where(cid == 0, sid * EPT0, 16 * EPT0 + sid * EPT1)
    nchunks = jnp.where(cid == 0, CPT0, CPT1)
    m0 = m_v[pl.ds(0, 16)][0]

    def chunk(c, carry):
        off = base + c * CHUNK
        pltpu.sync_copy(src_hbm.at[pl.ds(off, CHUNK)], src_v)
        pltpu.sync_copy(dst_hbm.at[pl.ds(off, CHUNK)], dst_v)
        cp = pltpu.async_copy(he_hbm.at[src_v], rows_v, sem)
        for k in range(CHUNK // 16):
            sv = src_v[pl.ds(k * 16, 16)]
            dv = dst_v[pl.ds(k * 16, 16)]
            e = plsc.load_gather(as_v, [sv]) + plsc.load_gather(ad_v, [dv])
            e = jnp.where(e > 0, e, e * 0.2)
            ex_v[pl.ds(k * 16, 16)] = jnp.exp(e - m0)
        cp.wait()

        def scale(k, carry2):
            exv = ex_v[pl.ds(k * 16, 16)]
            for l in range(16):
                w = exv[l]
                i = k * 16 + l
                for j in range(WD // 16):
                    rows_v[i, pl.ds(j * 16, 16)] = (
                        rows_v[i, pl.ds(j * 16, 16)] * w)
            return carry2

        lax.fori_loop(0, CHUNK // 16, scale, 0)
        pltpu.sync_copy(rows_v, acc.at[dst_v], add=True)
        return carry

    lax.fori_loop(0, nchunks, chunk, 0)
    plsc.subcore_barrier()

    @pl.when(sid < 15)
    def _():
        pltpu.sync_copy(acc.at[pl.ds(row0, 624)],
                        part_hbm.at[cid, pl.ds(row0, 624)])

    @pl.when(sid == 15)
    def _():
        pltpu.sync_copy(acc.at[pl.ds(9360, 648)],
                        part_hbm.at[cid, pl.ds(9360, 648)])


def _sc_edge_pass(he, a_src, a_dst, m16, src_p, dst_p):
    mesh = plsc.VectorSubcoreMesh(core_axis_name="c", subcore_axis_name="s")
    k = functools.partial(
        pl.kernel,
        out_type=jax.ShapeDtypeStruct((2, NP, WD), jnp.float32),
        mesh=mesh,
        compiler_params=pltpu.CompilerParams(
            needs_layout_passes=False, use_tc_tiling_on_sc=False),
        scratch_types=[
            pltpu.VMEM_SHARED((NP, WD), jnp.float32),
            pltpu.VMEM((NP,), jnp.float32),
            pltpu.VMEM((NP,), jnp.float32),
            pltpu.VMEM((16,), jnp.float32),
            pltpu.VMEM((CHUNK,), jnp.int32),
            pltpu.VMEM((CHUNK,), jnp.int32),
            pltpu.VMEM((CHUNK, WD), jnp.float32),
            pltpu.VMEM((CHUNK,), jnp.float32),
            pltpu.SemaphoreType.DMA,
        ],
    )(_sc_body)
    return k(he, a_src, a_dst, m16, src_p, dst_p)


def _epi_body(part_ref, he_ref, asr, adr, m_ref, b_ref, bat_ref, out_ref):
    g = pl.program_id(0)
    acc = part_ref[0, :, pl.ds(0, 128)] + part_ref[1, :, pl.ds(0, 128)]
    den = part_ref[0, :, pl.ds(128, 1)] + part_ref[1, :, pl.ds(128, 1)]
    es = asr[:] + adr[:]
    es = jnp.where(es > 0, es, es * 0.2)
    exs = jnp.exp(es - m_ref[0, 0])
    tot = acc + exs * he_ref[:, pl.ds(0, 128)]
    dent = den + exs + 1e-16
    r = jnp.maximum(tot / dent + b_ref[:], 0.0)
    cur = jnp.where(g == 0, jnp.full((G, 128), -jnp.inf, jnp.float32), out_ref[:])
    rows = []
    for gg in range(G):
        rm = jnp.where(bat_ref[:] == gg, r, -jnp.inf)
        rows.append(jnp.max(rm, axis=0, keepdims=True))
    cur = jnp.maximum(cur, jnp.concatenate(rows, axis=0))

    @pl.when(g < GRID - 1)
    def _():
        out_ref[:] = cur

    @pl.when(g == GRID - 1)
    def _():
        out_ref[:] = jnp.where(jnp.isfinite(cur), cur, 0.0)


def _epilogue(part, he, a_src, a_dst, m128, bias, batch2d):
    return pl.pallas_call(
        _epi_body,
        grid=(GRID,),
        in_specs=[
            pl.BlockSpec((2, BN, WD), lambda g: (0, g, 0)),
            pl.BlockSpec((BN, WD), lambda g: (g, 0)),
            pl.BlockSpec((BN, 1), lambda g: (g, 0)),
            pl.BlockSpec((BN, 1), lambda g: (g, 0)),
            pl.BlockSpec((1, 128), lambda g: (0, 0)),
            pl.BlockSpec((1, 128), lambda g: (0, 0)),
            pl.BlockSpec((BN, 1), lambda g: (g, 0)),
        ],
        out_specs=pl.BlockSpec((G, 128), lambda g: (0, 0)),
        out_shape=jax.ShapeDtypeStruct((G, 128), jnp.float32),
    )(part, he, a_src, a_dst, m128, bias, batch2d)


def kernel(x, edge_index, batch, W, att_src, att_dst, bias):
    x_pad = jnp.concatenate([x, jnp.zeros((NP - N, D), jnp.float32)], axis=0)
    pad = jnp.full((EPAD - E,), N, jnp.int32)
    src_p = jnp.concatenate([edge_index[0], pad])
    dst_p = jnp.concatenate([edge_index[1], pad])
    batch2d = jnp.concatenate([batch, jnp.full((NP - N,), G, jnp.int32)]).reshape(NP, 1)

    he, a_src2d, a_dst2d, m128 = _pre(
        x_pad, W, att_src.reshape(1, 128), att_dst.reshape(1, 128))
    part = _sc_edge_pass(he, a_src2d.reshape(NP), a_dst2d.reshape(NP),
                         m128[0, :16], src_p, dst_p)
    return _epilogue(part, he, a_src2d, a_dst2d, m128,
                     bias.reshape(1, 128), batch2d)
